# baseline (speedup 1.0000x reference)
import functools

import jax
import jax.numpy as jnp
from jax import lax
from jax.experimental import pallas as pl
from jax.experimental.pallas import tpu as pltpu
from jax.experimental.pallas import tpu_sc as plsc

B = 4096
L = 200
DIM = 32
V = 1000000

_info = plsc.get_sparse_core_info()
_NC, _NS = _info.num_cores, _info.num_subcores
NW = _NC * _NS
BPW = B // NW

RN = 800
RP_NCH = V // RN
RP_SLOTS = -(-RP_NCH // NW)

_SC_PARAMS = pltpu.CompilerParams(
    use_tc_tiling_on_sc=False, needs_layout_passes=False)


def _iota16():
  return jax.lax.broadcasted_iota(jnp.int32, (16,), 0)


def _repack_and_context(qid, ctxF, exF):
  mesh = plsc.VectorSubcoreMesh(core_axis_name="c", subcore_axis_name="s")

  @functools.partial(
      pl.kernel,
      mesh=mesh,
      out_type=(jax.ShapeDtypeStruct((V, DIM), jnp.float32),
                jax.ShapeDtypeStruct((DIM, B), jnp.float32)),
      scratch_types=[
          pltpu.VMEM((DIM, RN), jnp.float32),
          pltpu.VMEM((DIM, RN), jnp.float32),
          pltpu.VMEM((RN, DIM), jnp.float32),
          pltpu.VMEM((BPW,), jnp.int32),
          pltpu.VMEM((DIM, BPW), jnp.int32),
          pltpu.VMEM((DIM, BPW), jnp.float32),
          pltpu.SemaphoreType.DMA((2,)),
          pltpu.SemaphoreType.DMA,
      ],
      compiler_params=_SC_PARAMS,
  )
  def k(qid_hbm, ctxF_hbm, exF_hbm, scr, ctxT_out,
        tbufA, tbufB, obuf, cidx_v, cgim, crows, rsem, csem):
    wid = lax.axis_index("s") * _NC + lax.axis_index("c")

    pltpu.sync_copy(qid_hbm.at[pl.ds(wid * BPW, BPW)], cidx_v)
    def cprep(c, carry):
      for g in range(BPW // 16):
        vec = cidx_v[pl.ds(g * 16, 16)]
        cgim[c, pl.ds(g * 16, 16)] = vec + c * V
      return carry
    lax.fori_loop(0, DIM, cprep, 0)
    for c in range(DIM):
      pltpu.async_copy(ctxF_hbm.at[cgim.at[c]], crows.at[c], csem)
    for c in range(DIM):
      pltpu.make_async_copy(ctxF_hbm.at[cgim.at[c]], crows.at[c],
                            csem).wait()
    pltpu.sync_copy(crows, ctxT_out.at[:, pl.ds(wid * BPW, BPW)])

    tbufs = (tbufA, tbufB)

    def start_read(c, b):
      for cc in range(DIM):
        pltpu.async_copy(exF_hbm.at[pl.ds(cc * V + c * RN, RN)],
                         tbufs[b].at[cc], rsem.at[b])

    def wait_read(c, b):
      for cc in range(DIM):
        pltpu.make_async_copy(exF_hbm.at[pl.ds(cc * V + c * RN, RN)],
                              tbufs[b].at[cc], rsem.at[b]).wait()

    def transpose(b):
      tb = tbufs[b]
      def rows8(i, carry):
        for u in range(8):
          r = i * 8 + u
          rv = jnp.full((16,), 0, jnp.int32) + r
          lo = plsc.load_gather(tb, [_iota16(), rv])
          hi = plsc.load_gather(tb, [_iota16() + 16, rv])
          obuf[r, pl.ds(0, 16)] = lo
          obuf[r, pl.ds(16, 16)] = hi
        return carry
      lax.fori_loop(0, RN // 8, rows8, 0)

    def write(c):
      pltpu.sync_copy(obuf, scr.at[pl.ds(c * RN, RN)])

    start_read(wid, 0)
    def group(g, carry):
      for s in range(2):
        j = g * 2 + s
        c = wid + j * NW
        @pl.when(c < RP_NCH)
        def _():
          nxt = c + NW
          @pl.when(nxt < RP_NCH)
          def _():
            start_read(nxt, 1 - s)
          wait_read(c, s)
          transpose(s)
          write(c)
      return carry
    lax.fori_loop(0, RP_SLOTS // 2, group, 0)

  return k(qid, ctxF, exF)


def _gather(didT, scr):
  mesh = plsc.VectorSubcoreMesh(core_axis_name="c", subcore_axis_name="s")

  @functools.partial(
      pl.kernel,
      mesh=mesh,
      out_type=jax.ShapeDtypeStruct((L, DIM, B), jnp.float32),
      scratch_types=[
          pltpu.VMEM((L, BPW), jnp.int32),
          pltpu.VMEM((BPW, DIM), jnp.float32),
          pltpu.VMEM((BPW, DIM), jnp.float32),
          pltpu.VMEM((DIM, BPW), jnp.float32),
          pltpu.VMEM((DIM, BPW), jnp.float32),
          pltpu.SemaphoreType.DMA((2,)),
      ],
      compiler_params=_SC_PARAMS,
  )
  def k(didT_hbm, scr_hbm, exT_out, idx_v, growA, growB, bbufA, bbufB,
        gsem):
    wid = lax.axis_index("s") * _NC + lax.axis_index("c")
    b0 = wid * BPW

    pltpu.sync_copy(didT_hbm.at[:, pl.ds(b0, BPW)], idx_v)

    grows = (growA, growB)
    bbufs = (bbufA, bbufB)

    def start_gather(l, b):
      pltpu.async_copy(scr_hbm.at[idx_v.at[l]], grows[b], gsem.at[b])

    def wait_gather(l, b):
      pltpu.make_async_copy(scr_hbm.at[idx_v.at[l]], grows[b],
                            gsem.at[b]).wait()

    def transpose(b):
      gr, bb = grows[b], bbufs[b]
      def cols(c, carry):
        cv = jnp.full((16,), 0, jnp.int32) + c
        for g in range(BPW // 16):
          vals = plsc.load_gather(gr, [_iota16() + g * 16, cv])
          bb[c, pl.ds(g * 16, 16)] = vals
        return carry
      lax.fori_loop(0, DIM, cols, 0)

    def write(l, b):
      pltpu.sync_copy(bbufs[b], exT_out.at[l, :, pl.ds(b0, BPW)])

    start_gather(0, 0)
    def group(g, carry):
      for s in range(2):
        l = g * 2 + s
        @pl.when(l + 1 < L)
        def _():
          start_gather(l + 1, 1 - s)
        wait_gather(l, s)
        transpose(s)
        write(l, s)
      return carry
    lax.fori_loop(0, L // 2, group, 0)

  return k(didT, scr)


def kernel(query_id, doc_id, context_table, example_table):
  qid = query_id.astype(jnp.int32)
  didT = doc_id.astype(jnp.int32).T
  ctxF = context_table.T.reshape(-1)
  exF = example_table.T.reshape(-1)
  scr, ctxT = _repack_and_context(qid, ctxF, exF)
  exT_out = _gather(didT, scr)
  return (ctxT.T, jnp.transpose(exT_out, (2, 0, 1)))


# speedup vs baseline: 6.5915x; 6.5915x over previous
import functools

import jax
import jax.numpy as jnp
from jax import lax
from jax.experimental import pallas as pl
from jax.experimental.pallas import tpu as pltpu
from jax.experimental.pallas import tpu_sc as plsc

B = 4096
L = 200
DIM = 32
V = 1000000

_info = plsc.get_sparse_core_info()
_NC, _NS = _info.num_cores, _info.num_subcores
NW = _NC * _NS
TOTAL = B * L
ROWS_PER_W = TOTAL // NW
CHUNK = 128
NCHUNK = ROWS_PER_W // CHUNK
CTX_PER_W = B // NW
K = 8
D = 4


def _iota16():
  return jax.lax.broadcasted_iota(jnp.int32, (16,), 0)


def _context(qid, ctxT):
  mesh = plsc.VectorSubcoreMesh(core_axis_name="c", subcore_axis_name="s")

  @functools.partial(
      pl.kernel,
      mesh=mesh,
      out_type=jax.ShapeDtypeStruct((DIM, B), jnp.float32),
      scratch_types=[
          pltpu.VMEM((CTX_PER_W,), jnp.int32),
          pltpu.VMEM((2, DIM, 128), jnp.float32),
          pltpu.VMEM((DIM, CTX_PER_W), jnp.float32),
          pltpu.SemaphoreType.DMA((2,)),
      ],
      compiler_params=pltpu.CompilerParams(needs_layout_passes=False),
  )
  def k(qid_hbm, ctxT_hbm, ctxT_out, cidx_v, tbuf, crows, sem):
    wid = lax.axis_index("s") * _NC + lax.axis_index("c")
    pltpu.sync_copy(qid_hbm.at[wid], cidx_v)

    def fetch(q, b):
      rt = lax.shift_right_logical(q, 7)
      for cc in range(DIM // 8):
        pltpu.async_copy(
            ctxT_hbm.at[pl.ds(cc * 8, 8), pl.ds(rt * 128, 128)],
            tbuf.at[b, pl.ds(cc * 8, 8)], sem.at[b])

    def drain(q, b):
      rt = lax.shift_right_logical(q, 7)
      for cc in range(DIM // 8):
        pltpu.make_async_copy(
            ctxT_hbm.at[pl.ds(cc * 8, 8), pl.ds(rt * 128, 128)],
            tbuf.at[b, pl.ds(cc * 8, 8)], sem.at[b]).wait()

    def extract(q, b, i):
      col = lax.bitwise_and(q, jnp.int32(127))
      colv = jnp.full((16,), 0, jnp.int32) + col
      iv = jnp.full((16,), 0, jnp.int32) + i
      for h in range(2):
        rows = _iota16() + h * 16
        vals = plsc.load_gather(tbuf.at[b], [rows, colv])
        plsc.store_scatter(crows, [rows, iv], vals)

    def grp(g, carry):
      vec = cidx_v[pl.ds(g * 16, 16)]
      for i in range(16):
        q = vec[i]
        b = i % 2
        fetch(q, b)
        drain(q, b)
        extract(q, b, g * 16 + i)
      return carry
    lax.fori_loop(0, CTX_PER_W // 16, grp, 0)

    pltpu.sync_copy(crows, ctxT_out.at[:, pl.ds(wid * CTX_PER_W, CTX_PER_W)])

  return k(qid, ctxT)


def _gather(did, ex_tab):
  mesh = plsc.VectorSubcoreMesh(core_axis_name="c", subcore_axis_name="s")

  @functools.partial(
      pl.kernel,
      mesh=mesh,
      out_type=jax.ShapeDtypeStruct((TOTAL, DIM), jnp.float32),
      scratch_types=[
          pltpu.VMEM((NCHUNK, CHUNK), jnp.int32),
          pltpu.VMEM((K, CHUNK, DIM), jnp.float32),
          pltpu.SemaphoreType.DMA((K,)),
          pltpu.SemaphoreType.DMA((K,)),
      ],
      compiler_params=pltpu.CompilerParams(use_tc_tiling_on_sc=False),
  )
  def k(did_hbm, ex_hbm, ex_out, idx_v, rows_v, gsem, wsem):
    wid = lax.axis_index("s") * _NC + lax.axis_index("c")
    base = wid * ROWS_PER_W

    def start_gather(j, b):
      pltpu.async_copy(ex_hbm.at[idx_v.at[j]], rows_v.at[b], gsem.at[b])

    def wait_gather(j, b):
      pltpu.make_async_copy(
          ex_hbm.at[idx_v.at[j]], rows_v.at[b], gsem.at[b]).wait()

    def start_write(j, b):
      pltpu.async_copy(
          rows_v.at[b], ex_out.at[pl.ds(base + j * CHUNK, CHUNK)], wsem.at[b])

    def wait_write(j, b):
      pltpu.make_async_copy(
          rows_v.at[b], ex_out.at[pl.ds(base + j * CHUNK, CHUNK)],
          wsem.at[b]).wait()

    pltpu.sync_copy(did_hbm.at[wid], idx_v)

    for j in range(D):
      start_gather(j, j % K)
    for j in range(K - D):
      wait_gather(j, j % K)
      start_write(j, j % K)
      start_gather(j + D, (j + D) % K)

    def group(g, carry):
      for b in range(K):
        j = (K - D) + g * K + b
        bj = (K - D + b) % K
        bf = b
        wait_gather(j, bj)
        start_write(j, bj)
        wait_write(j + D - K, bf)
        start_gather(j + D, bf)
      return carry
    lax.fori_loop(0, (NCHUNK - K) // K, group, 0)

    for j in range(NCHUNK - D, NCHUNK):
      wait_gather(j, j % K)
      start_write(j, j % K)
    for j in range(NCHUNK - K, NCHUNK):
      wait_write(j, j % K)

  return k(did, ex_tab)


def kernel(query_id, doc_id, context_table, example_table):
  qid = query_id.astype(jnp.int32).reshape(NW, CTX_PER_W)
  did = doc_id.astype(jnp.int32).reshape(NW, NCHUNK, CHUNK)
  ctxT_out = _context(qid, context_table.T)
  ex_flat = _gather(did, example_table)
  return (ctxT_out.T, ex_flat.reshape(B, L, DIM))


# speedup vs baseline: 7.4961x; 1.1372x over previous
import functools

import jax
import jax.numpy as jnp
from jax import lax
from jax.experimental import pallas as pl
from jax.experimental.pallas import tpu as pltpu
from jax.experimental.pallas import tpu_sc as plsc

B = 4096
L = 200
DIM = 32
V = 1000000

_info = plsc.get_sparse_core_info()
_NC, _NS = _info.num_cores, _info.num_subcores
NW = _NC * _NS
BPW = B // NW
CTX_PER_W = B // NW
BBS = BPW + 1


def _iota16():
  return jax.lax.broadcasted_iota(jnp.int32, (16,), 0)


def _context(qid, ctxT):
  mesh = plsc.VectorSubcoreMesh(core_axis_name="c", subcore_axis_name="s")

  @functools.partial(
      pl.kernel,
      mesh=mesh,
      out_type=jax.ShapeDtypeStruct((DIM, B), jnp.float32),
      scratch_types=[
          pltpu.VMEM((CTX_PER_W,), jnp.int32),
          pltpu.VMEM((2, DIM, 128), jnp.float32),
          pltpu.VMEM((DIM, CTX_PER_W), jnp.float32),
          pltpu.SemaphoreType.DMA((2,)),
      ],
      compiler_params=pltpu.CompilerParams(needs_layout_passes=False),
  )
  def k(qid_hbm, ctxT_hbm, ctxT_out, cidx_v, tbuf, crows, sem):
    wid = lax.axis_index("s") * _NC + lax.axis_index("c")
    pltpu.sync_copy(qid_hbm.at[wid], cidx_v)

    def fetch(q, b):
      rt = lax.shift_right_logical(q, 7)
      for cc in range(DIM // 8):
        pltpu.async_copy(
            ctxT_hbm.at[pl.ds(cc * 8, 8), pl.ds(rt * 128, 128)],
            tbuf.at[b, pl.ds(cc * 8, 8)], sem.at[b])

    def drain(q, b):
      rt = lax.shift_right_logical(q, 7)
      for cc in range(DIM // 8):
        pltpu.make_async_copy(
            ctxT_hbm.at[pl.ds(cc * 8, 8), pl.ds(rt * 128, 128)],
            tbuf.at[b, pl.ds(cc * 8, 8)], sem.at[b]).wait()

    def extract(q, b, i):
      col = lax.bitwise_and(q, jnp.int32(127))
      colv = jnp.full((16,), 0, jnp.int32) + col
      iv = jnp.full((16,), 0, jnp.int32) + i
      for h in range(2):
        rows = _iota16() + h * 16
        vals = plsc.load_gather(tbuf.at[b], [rows, colv])
        plsc.store_scatter(crows, [rows, iv], vals)

    def grp(g, carry):
      vec = cidx_v[pl.ds(g * 16, 16)]
      for i in range(16):
        q = vec[i]
        b = i % 2
        fetch(q, b)
        drain(q, b)
        extract(q, b, g * 16 + i)
      return carry
    lax.fori_loop(0, CTX_PER_W // 16, grp, 0)

    pltpu.sync_copy(crows, ctxT_out.at[:, pl.ds(wid * CTX_PER_W, CTX_PER_W)])

  return k(qid, ctxT)


def _gather(didT, ex_tab):
  mesh = plsc.VectorSubcoreMesh(core_axis_name="c", subcore_axis_name="s")

  @functools.partial(
      pl.kernel,
      mesh=mesh,
      out_type=jax.ShapeDtypeStruct((L, DIM, B), jnp.float32),
      scratch_types=[
          pltpu.VMEM((L, BPW), jnp.int32),
          pltpu.VMEM((2, BPW, DIM), jnp.float32),
          pltpu.VMEM((2, DIM, BBS), jnp.float32),
          pltpu.SemaphoreType.DMA((2,)),
      ],
      compiler_params=pltpu.CompilerParams(
          use_tc_tiling_on_sc=False, needs_layout_passes=False),
  )
  def k(didT_hbm, ex_hbm, exT_out, idx_v, grows, bbufs, gsem):
    wid = lax.axis_index("s") * _NC + lax.axis_index("c")
    b0 = wid * BPW

    pltpu.sync_copy(didT_hbm.at[:, pl.ds(b0, BPW)], idx_v)

    def start_gather(l, b):
      pltpu.async_copy(ex_hbm.at[idx_v.at[l]], grows.at[b], gsem.at[b])

    def wait_gather(l, b):
      pltpu.make_async_copy(ex_hbm.at[idx_v.at[l]], grows.at[b],
                            gsem.at[b]).wait()

    def transpose(b):
      def rows4(i, carry):
        for u in range(4):
          r = i * 4 + u
          rv = jnp.full((16,), 0, jnp.int32) + r
          for h in range(2):
            rows = _iota16() + h * 16
            vals = grows[b, r, pl.ds(h * 16, 16)]
            plsc.store_scatter(bbufs.at[b], [rows, rv], vals)
        return carry
      lax.fori_loop(0, BPW // 4, rows4, 0)

    def write(l, b):
      pltpu.sync_copy(bbufs.at[b, :, pl.ds(0, BPW)],
                      exT_out.at[l, :, pl.ds(b0, BPW)])

    start_gather(0, 0)
    def group(g, carry):
      for s in range(2):
        l = g * 2 + s
        @pl.when(l + 1 < L)
        def _():
          start_gather(l + 1, 1 - s)
        wait_gather(l, s)
        transpose(s)
        write(l, s)
      return carry
    lax.fori_loop(0, L // 2, group, 0)

  return k(didT, ex_tab)


def kernel(query_id, doc_id, context_table, example_table):
  qid = query_id.astype(jnp.int32).reshape(NW, CTX_PER_W)
  didT = doc_id.astype(jnp.int32).T
  ctxT_out = _context(qid, context_table.T)
  exT_out = _gather(didT, example_table)
  return (ctxT_out.T, jnp.transpose(exT_out, (2, 0, 1)))


# speedup vs baseline: 7.5092x; 1.0018x over previous
import functools

import jax
import jax.numpy as jnp
from jax import lax
from jax.experimental import pallas as pl
from jax.experimental.pallas import tpu as pltpu
from jax.experimental.pallas import tpu_sc as plsc

B = 4096
L = 200
DIM = 32
V = 1000000

_info = plsc.get_sparse_core_info()
_NC, _NS = _info.num_cores, _info.num_subcores
NW = _NC * _NS
BPW = B // NW
CTX_PER_W = B // NW
BBS = BPW + 1


def _iota16():
  return jax.lax.broadcasted_iota(jnp.int32, (16,), 0)


def _context(qid, ctxT):
  mesh = plsc.VectorSubcoreMesh(core_axis_name="c", subcore_axis_name="s")

  @functools.partial(
      pl.kernel,
      mesh=mesh,
      out_type=jax.ShapeDtypeStruct((DIM, B), jnp.float32),
      scratch_types=[
          pltpu.VMEM((CTX_PER_W,), jnp.int32),
          pltpu.VMEM((2, DIM, 128), jnp.float32),
          pltpu.VMEM((DIM, CTX_PER_W), jnp.float32),
          pltpu.SemaphoreType.DMA((2,)),
      ],
      compiler_params=pltpu.CompilerParams(needs_layout_passes=False),
  )
  def k(qid_hbm, ctxT_hbm, ctxT_out, cidx_v, tbuf, crows, sem):
    wid = lax.axis_index("s") * _NC + lax.axis_index("c")
    pltpu.sync_copy(qid_hbm.at[wid], cidx_v)

    def fetch(q, b):
      rt = lax.shift_right_logical(q, 7)
      for cc in range(DIM // 8):
        pltpu.async_copy(
            ctxT_hbm.at[pl.ds(cc * 8, 8), pl.ds(rt * 128, 128)],
            tbuf.at[b, pl.ds(cc * 8, 8)], sem.at[b])

    def drain(q, b):
      rt = lax.shift_right_logical(q, 7)
      for cc in range(DIM // 8):
        pltpu.make_async_copy(
            ctxT_hbm.at[pl.ds(cc * 8, 8), pl.ds(rt * 128, 128)],
            tbuf.at[b, pl.ds(cc * 8, 8)], sem.at[b]).wait()

    def extract(q, b, i):
      col = lax.bitwise_and(q, jnp.int32(127))
      colv = jnp.full((16,), 0, jnp.int32) + col
      iv = jnp.full((16,), 0, jnp.int32) + i
      for h in range(2):
        rows = _iota16() + h * 16
        vals = plsc.load_gather(tbuf.at[b], [rows, colv])
        plsc.store_scatter(crows, [rows, iv], vals)

    def grp(g, carry):
      vec = cidx_v[pl.ds(g * 16, 16)]
      for i in range(16):
        q = vec[i]
        b = i % 2
        fetch(q, b)
        drain(q, b)
        extract(q, b, g * 16 + i)
      return carry
    lax.fori_loop(0, CTX_PER_W // 16, grp, 0)

    pltpu.sync_copy(crows, ctxT_out.at[:, pl.ds(wid * CTX_PER_W, CTX_PER_W)])

  return k(qid, ctxT)


def _gather(did4, ex_tab):
  mesh = plsc.VectorSubcoreMesh(core_axis_name="c", subcore_axis_name="s")

  @functools.partial(
      pl.kernel,
      mesh=mesh,
      out_type=jax.ShapeDtypeStruct((L, DIM, B), jnp.float32),
      scratch_types=[
          pltpu.VMEM((L // 8, 1, 8, BPW), jnp.int32),
          pltpu.VMEM((2, BPW, DIM), jnp.float32),
          pltpu.VMEM((2, DIM, BBS), jnp.float32),
          pltpu.SemaphoreType.DMA((2,)),
      ],
      compiler_params=pltpu.CompilerParams(
          use_tc_tiling_on_sc=False, needs_layout_passes=False),
  )
  def k(did4_hbm, ex_hbm, exT_out, idx_v, grows, bbufs, gsem):
    wid = lax.axis_index("s") * _NC + lax.axis_index("c")
    b0 = wid * BPW

    pltpu.sync_copy(did4_hbm.at[:, pl.ds(wid, 1)], idx_v)

    def start_gather(l, b):
      pltpu.async_copy(ex_hbm.at[idx_v.at[l // 8, 0, l % 8]],
                       grows.at[b], gsem.at[b])

    def wait_gather(l, b):
      pltpu.make_async_copy(ex_hbm.at[idx_v.at[l // 8, 0, l % 8]],
                            grows.at[b], gsem.at[b]).wait()

    def transpose(b):
      def rows4(i, carry):
        for u in range(4):
          r = i * 4 + u
          rv = jnp.full((16,), 0, jnp.int32) + r
          for h in range(2):
            rows = _iota16() + h * 16
            vals = grows[b, r, pl.ds(h * 16, 16)]
            plsc.store_scatter(bbufs.at[b], [rows, rv], vals)
        return carry
      lax.fori_loop(0, BPW // 4, rows4, 0)

    def write(l, b):
      pltpu.sync_copy(bbufs.at[b, :, pl.ds(0, BPW)],
                      exT_out.at[l, :, pl.ds(b0, BPW)])

    start_gather(0, 0)
    def group(g, carry):
      for s in range(2):
        l = g * 2 + s
        @pl.when(l + 1 < L)
        def _():
          start_gather(l + 1, 1 - s)
        wait_gather(l, s)
        transpose(s)
        write(l, s)
      return carry
    lax.fori_loop(0, L // 2, group, 0)

  return k(did4, ex_tab)


def kernel(query_id, doc_id, context_table, example_table):
  qid = query_id.astype(jnp.int32).reshape(NW, CTX_PER_W)
  did4 = (doc_id.astype(jnp.int32).T
          .reshape(L // 8, 8, NW, BPW).transpose(0, 2, 1, 3))
  ctxT_out = _context(qid, context_table.T)
  exT_out = _gather(did4, example_table)
  return (ctxT_out.T, jnp.transpose(exT_out, (2, 0, 1)))
